# baseline (speedup 1.0000x reference)
import functools
import jax
import jax.numpy as jnp
import numpy as np
from jax.experimental import pallas as pl
from jax.experimental.pallas import tpu as pltpu


def _mlp(ps, h):
    n = len(ps)
    for i, (W, b) in enumerate(ps):
        h = h @ W + b
        if i < n - 1:
            h = jax.nn.relu(h)
    return h


def _fps(pos, npoints):
    N = pos.shape[0]

    def body(i, state):
        sel, mind, last = state
        d = jnp.sum((pos - pos[last]) ** 2, axis=-1)
        mind = jnp.minimum(mind, d)
        nxt = jnp.argmax(mind).astype(jnp.int32)
        sel = sel.at[i].set(nxt)
        return (sel, mind, nxt)

    sel0 = jnp.zeros((npoints,), dtype=jnp.int32)
    init = (sel0, jnp.full((N,), jnp.inf, dtype=pos.dtype), jnp.int32(0))
    sel, _, _ = jax.lax.fori_loop(1, npoints, body, init)
    return sel


def _sa(x, pos, ratio, r, ps, K=64):
    N = pos.shape[0]
    M = int(N * ratio)
    idx_c = _fps(pos, M)
    pos_c = pos[idx_c]
    d2 = jnp.sum((pos_c[:, None, :] - pos[None, :, :]) ** 2, axis=-1)
    score = jnp.where(d2 <= r * r, -d2, -jnp.inf)
    svals, nidx = jax.lax.top_k(score, K)
    valid = svals > -jnp.inf
    xj = x[nidx]
    pj = pos[nidx] - pos_c[:, None, :]
    h = _mlp(ps, jnp.concatenate([xj, pj], axis=-1))
    h = jnp.where(valid[:, :, None], h, -jnp.inf)
    return jnp.max(h, axis=1), pos_c


def _knn_interp(x, pos_src, pos_tgt, k):
    d2 = jnp.sum((pos_tgt[:, None, :] - pos_src[None, :, :]) ** 2, axis=-1)
    _, idx = jax.lax.top_k(-d2, k)
    d2k = jnp.take_along_axis(d2, idx, axis=1)
    w = 1.0 / jnp.maximum(d2k, 1e-16)
    y = jnp.sum(w[:, :, None] * x[idx], axis=1) / jnp.sum(w, axis=1, keepdims=True)
    return y


def _head_body(y_ref, w1_ref, b1_ref, w2_ref, b2_ref, w3_ref, b3_ref, o_ref):
    h = y_ref[...]
    h = jnp.maximum(h @ w1_ref[...] + b1_ref[...], 0.0)
    h = jnp.maximum(h @ w2_ref[...] + b2_ref[...], 0.0)
    o_ref[...] = h @ w3_ref[...] + b3_ref[...]


def _head(y, ps):
    (w1, b1), (w2, b2), (w3, b3) = ps
    R = y.shape[0]
    BR = 1024
    grid = (R // BR,)
    out = pl.pallas_call(
        _head_body,
        grid=grid,
        in_specs=[
            pl.BlockSpec((BR, 128), lambda i: (i, 0)),
            pl.BlockSpec((128, 128), lambda i: (0, 0)),
            pl.BlockSpec((128,), lambda i: (0,)),
            pl.BlockSpec((128, 128), lambda i: (0, 0)),
            pl.BlockSpec((128,), lambda i: (0,)),
            pl.BlockSpec((128, 8), lambda i: (0, 0)),
            pl.BlockSpec((8,), lambda i: (0,)),
        ],
        out_specs=pl.BlockSpec((BR, 8), lambda i: (i, 0)),
        out_shape=jax.ShapeDtypeStruct((R, 8), jnp.float32),
    )(y, w1, b1, w2, b2, w3, b3)
    return out[:, :3]


def kernel(x, pos, batch, params):
    x1, pos1 = _sa(x, pos, 0.5, 0.2, params['sa1'])
    x2, pos2 = _sa(x1, pos1, 0.25, 0.4, params['sa2'])
    h3 = _mlp(params['sa3'], jnp.concatenate([x2, pos2], axis=-1))
    x3 = jnp.max(h3, axis=0, keepdims=True)
    y = jnp.broadcast_to(x3, (x2.shape[0], x3.shape[1]))
    y = _mlp(params['fp3'], jnp.concatenate([y, x2], axis=-1))
    y = _knn_interp(y, pos2, pos1, 3)
    y = _mlp(params['fp2'], jnp.concatenate([y, x1], axis=-1))
    y = _knn_interp(y, pos1, pos, 3)
    y = _mlp(params['fp1'], jnp.concatenate([y, x], axis=-1))
    (w3, b3) = params['head'][2]
    w3p = jnp.pad(w3, ((0, 0), (0, 5)))
    b3p = jnp.pad(b3, ((0, 5),))
    head_ps = [params['head'][0], params['head'][1], (w3p, b3p)]
    return _head(y, head_ps)


# speedup vs baseline: 2.2857x; 2.2857x over previous
import functools
import jax
import jax.numpy as jnp
import numpy as np
from jax.experimental import pallas as pl
from jax.experimental.pallas import tpu as pltpu


def _mlp(ps, h):
    n = len(ps)
    for i, (W, b) in enumerate(ps):
        h = h @ W + b
        if i < n - 1:
            h = jax.nn.relu(h)
    return h


def _fps_body(px_ref, py_ref, pz_ref, pc_ref, *, n, m, cols):
    rows = n // cols
    ri = jax.lax.broadcasted_iota(jnp.int32, (rows, cols), 0)
    ci = jax.lax.broadcasted_iota(jnp.int32, (rows, cols), 1)
    flat = ri * cols + ci
    lane = jax.lax.broadcasted_iota(jnp.int32, (1, 128), 1)

    px = px_ref[...]
    py = py_ref[...]
    pz = pz_ref[...]

    def write_row(i, qx, qy, qz):
        row = (jnp.where(lane == 0, qx, 0.0)
               + jnp.where(lane == 1, qy, 0.0)
               + jnp.where(lane == 2, qz, 0.0))
        pc_ref[pl.ds(i, 1), :] = row

    qx0 = px_ref[0, 0]
    qy0 = py_ref[0, 0]
    qz0 = pz_ref[0, 0]
    write_row(0, qx0, qy0, qz0)

    def body(i, state):
        mind, qx, qy, qz = state
        d = (px - qx) ** 2 + (py - qy) ** 2 + (pz - qz) ** 2
        mind = jnp.minimum(mind, d)
        mval = jnp.max(mind)
        idx = jnp.min(jnp.where(mind == mval, flat, jnp.int32(2 ** 30)))
        sel = flat == idx
        nqx = jnp.sum(jnp.where(sel, px, 0.0))
        nqy = jnp.sum(jnp.where(sel, py, 0.0))
        nqz = jnp.sum(jnp.where(sel, pz, 0.0))
        write_row(i, nqx, nqy, nqz)
        return (mind, nqx, nqy, nqz)

    mind0 = jnp.full((rows, cols), jnp.inf, dtype=jnp.float32)
    jax.lax.fori_loop(1, m, body, (mind0, qx0, qy0, qz0))


def _fps_pallas(pos, m):
    n = pos.shape[0]
    cols = 1024
    rows = n // cols
    px = pos[:, 0].reshape(rows, cols)
    py = pos[:, 1].reshape(rows, cols)
    pz = pos[:, 2].reshape(rows, cols)
    body = functools.partial(_fps_body, n=n, m=m, cols=cols)
    pc = pl.pallas_call(
        body,
        out_shape=jax.ShapeDtypeStruct((m, 128), jnp.float32),
    )(px, py, pz)
    return pc[:, :3]


def _sa(x, pos, ratio, r, ps, K=64):
    N = pos.shape[0]
    M = int(N * ratio)
    pos_c = _fps_pallas(pos, M)
    d2 = jnp.sum((pos_c[:, None, :] - pos[None, :, :]) ** 2, axis=-1)
    score = jnp.where(d2 <= r * r, -d2, -jnp.inf)
    svals, nidx = jax.lax.top_k(score, K)
    valid = svals > -jnp.inf
    xj = x[nidx]
    pj = pos[nidx] - pos_c[:, None, :]
    h = _mlp(ps, jnp.concatenate([xj, pj], axis=-1))
    h = jnp.where(valid[:, :, None], h, -jnp.inf)
    return jnp.max(h, axis=1), pos_c


def _knn_interp(x, pos_src, pos_tgt, k):
    d2 = jnp.sum((pos_tgt[:, None, :] - pos_src[None, :, :]) ** 2, axis=-1)
    _, idx = jax.lax.top_k(-d2, k)
    d2k = jnp.take_along_axis(d2, idx, axis=1)
    w = 1.0 / jnp.maximum(d2k, 1e-16)
    y = jnp.sum(w[:, :, None] * x[idx], axis=1) / jnp.sum(w, axis=1, keepdims=True)
    return y


def _head_body(y_ref, w1_ref, b1_ref, w2_ref, b2_ref, w3_ref, b3_ref, o_ref):
    h = y_ref[...]
    h = jnp.maximum(h @ w1_ref[...] + b1_ref[...], 0.0)
    h = jnp.maximum(h @ w2_ref[...] + b2_ref[...], 0.0)
    o_ref[...] = h @ w3_ref[...] + b3_ref[...]


def _head(y, ps):
    (w1, b1), (w2, b2), (w3, b3) = ps
    R = y.shape[0]
    BR = 1024
    grid = (R // BR,)
    out = pl.pallas_call(
        _head_body,
        grid=grid,
        in_specs=[
            pl.BlockSpec((BR, 128), lambda i: (i, 0)),
            pl.BlockSpec((128, 128), lambda i: (0, 0)),
            pl.BlockSpec((128,), lambda i: (0,)),
            pl.BlockSpec((128, 128), lambda i: (0, 0)),
            pl.BlockSpec((128,), lambda i: (0,)),
            pl.BlockSpec((128, 8), lambda i: (0, 0)),
            pl.BlockSpec((8,), lambda i: (0,)),
        ],
        out_specs=pl.BlockSpec((BR, 8), lambda i: (i, 0)),
        out_shape=jax.ShapeDtypeStruct((R, 8), jnp.float32),
    )(y, w1, b1, w2, b2, w3, b3)
    return out[:, :3]


def kernel(x, pos, batch, params):
    x1, pos1 = _sa(x, pos, 0.5, 0.2, params['sa1'])
    x2, pos2 = _sa(x1, pos1, 0.25, 0.4, params['sa2'])
    h3 = _mlp(params['sa3'], jnp.concatenate([x2, pos2], axis=-1))
    x3 = jnp.max(h3, axis=0, keepdims=True)
    y = jnp.broadcast_to(x3, (x2.shape[0], x3.shape[1]))
    y = _mlp(params['fp3'], jnp.concatenate([y, x2], axis=-1))
    y = _knn_interp(y, pos2, pos1, 3)
    y = _mlp(params['fp2'], jnp.concatenate([y, x1], axis=-1))
    y = _knn_interp(y, pos1, pos, 3)
    y = _mlp(params['fp1'], jnp.concatenate([y, x], axis=-1))
    (w3, b3) = params['head'][2]
    w3p = jnp.pad(w3, ((0, 0), (0, 5)))
    b3p = jnp.pad(b3, ((0, 5),))
    head_ps = [params['head'][0], params['head'][1], (w3p, b3p)]
    return _head(y, head_ps)


# speedup vs baseline: 7.4765x; 3.2709x over previous
import functools
import jax
import jax.numpy as jnp
import numpy as np
from jax import lax
from jax.experimental import pallas as pl
from jax.experimental.pallas import tpu as pltpu
from jax.experimental.pallas import tpu_sc as plsc


def _fps_body(px_ref, py_ref, pz_ref, pc_ref, *, n, m, cols):
    rows = n // cols
    ri = lax.broadcasted_iota(jnp.int32, (rows, cols), 0)
    ci = lax.broadcasted_iota(jnp.int32, (rows, cols), 1)
    flat = ri * cols + ci
    lane = lax.broadcasted_iota(jnp.int32, (1, 128), 1)

    px = px_ref[...]
    py = py_ref[...]
    pz = pz_ref[...]

    def write_row(i, qx, qy, qz):
        row = (jnp.where(lane == 0, qx, 0.0)
               + jnp.where(lane == 1, qy, 0.0)
               + jnp.where(lane == 2, qz, 0.0))
        pc_ref[pl.ds(i, 1), :] = row

    qx0 = px_ref[0, 0]
    qy0 = py_ref[0, 0]
    qz0 = pz_ref[0, 0]
    write_row(0, qx0, qy0, qz0)

    def body(i, state):
        mind, qx, qy, qz = state
        d = (px - qx) ** 2 + (py - qy) ** 2 + (pz - qz) ** 2
        mind = jnp.minimum(mind, d)
        mval = jnp.max(mind)
        idx = jnp.min(jnp.where(mind == mval, flat, jnp.int32(2 ** 30)))
        sel = flat == idx
        nqx = jnp.sum(jnp.where(sel, px, 0.0))
        nqy = jnp.sum(jnp.where(sel, py, 0.0))
        nqz = jnp.sum(jnp.where(sel, pz, 0.0))
        write_row(i, nqx, nqy, nqz)
        return (mind, nqx, nqy, nqz)

    mind0 = jnp.full((rows, cols), jnp.inf, dtype=jnp.float32)
    lax.fori_loop(1, m, body, (mind0, qx0, qy0, qz0))


def _fps_pallas(posx, posy, posz, m):
    rows, cols = posx.shape
    n = rows * cols
    body = functools.partial(_fps_body, n=n, m=m, cols=cols)
    pc = pl.pallas_call(
        body,
        out_shape=jax.ShapeDtypeStruct((m, 128), jnp.float32),
    )(posx, posy, posz)
    return pc


def _thresh_body(pc_ref, px_ref, py_ref, pz_ref, d2_ref, *, r2, kk, bm):
    cx = pc_ref[:, 0:1]
    cy = pc_ref[:, 1:2]
    cz = pc_ref[:, 2:3]
    d2 = (cx - px_ref[...]) ** 2 + (cy - py_ref[...]) ** 2 + (cz - pz_ref[...]) ** 2

    r2bits = jnp.int32(np.float32(r2).view(np.int32))
    cnt_r = jnp.sum(jnp.where(d2 <= r2, 1.0, 0.0), axis=1, keepdims=True)

    lo0 = jnp.zeros((bm, 1), jnp.int32)
    hi0 = jnp.full((bm, 1), r2bits, jnp.int32)

    def bit_body(_, state):
        lo, hi = state
        mid = lax.div(lo + hi, jnp.int32(2))
        tmid = pltpu.bitcast(mid, jnp.float32)
        cnt = jnp.sum(jnp.where(d2 <= tmid, 1.0, 0.0), axis=1, keepdims=True)
        ge = cnt >= float(kk)
        hi = jnp.where(ge, mid, hi)
        lo = jnp.where(ge, lo, mid + 1)
        return lo, hi

    lo, hi = lax.fori_loop(0, 31, bit_body, (lo0, hi0))
    t = pltpu.bitcast(hi, jnp.float32)
    t = jnp.where(cnt_r >= float(kk), t, jnp.float32(r2))
    d2_ref[...] = d2 - t


def _thresh_pallas(pc128, posx1, posy1, posz1, r2, kk=64, bm=256):
    m = pc128.shape[0]
    n = posx1.shape[1]
    body = functools.partial(_thresh_body, r2=r2, kk=kk, bm=bm)
    d2 = pl.pallas_call(
        body,
        grid=(m // bm,),
        in_specs=[
            pl.BlockSpec((bm, 128), lambda i: (i, 0)),
            pl.BlockSpec((1, n), lambda i: (0, 0)),
            pl.BlockSpec((1, n), lambda i: (0, 0)),
            pl.BlockSpec((1, n), lambda i: (0, 0)),
        ],
        out_specs=pl.BlockSpec((bm, n), lambda i: (i, 0)),
        out_shape=jax.ShapeDtypeStruct((m, n), jnp.float32),
    )(pc128, posx1, posy1, posz1)
    return d2


def _sc_group(d2, table, k=64):
    m, n = d2.shape
    wd = table.shape[1]
    nw = 32
    rpw = m // nw
    mesh = plsc.VectorSubcoreMesh(core_axis_name="c", subcore_axis_name="s")

    @functools.partial(
        pl.kernel,
        mesh=mesh,
        compiler_params=pltpu.CompilerParams(needs_layout_passes=False,
                                             use_tc_tiling_on_sc=False),
        out_type=[
            jax.ShapeDtypeStruct((m * k, wd), jnp.float32),
            jax.ShapeDtypeStruct((m,), jnp.int32),
        ],
        scratch_types=[
            pltpu.VMEM((n,), jnp.float32),
            pltpu.VMEM((k,), jnp.int32),
            pltpu.VMEM((k, wd), jnp.float32),
            pltpu.VMEM((rpw,), jnp.int32),
            pltpu.SemaphoreType.DMA,
        ],
    )
    def kfn(d2_hbm, table_hbm, g_hbm, cnt_hbm,
            d2row, idxrow, rows_v, cntbuf, sem):
        wid = lax.axis_index("s") * 2 + lax.axis_index("c")
        base = wid * rpw
        lanes = lax.broadcasted_iota(jnp.int32, (16,), 0)
        z16 = jnp.zeros((16,), jnp.int32)

        def row_body(rl, carry):
            row = base + rl
            pltpu.sync_copy(d2_hbm.at[row], d2row)
            for q in range(k // 16):
                idxrow[pl.ds(q * 16, 16)] = z16

            def chunk(j, off):
                v = d2row[pl.ds(j * 16, 16)]
                msk = v <= 0.0
                mi = jnp.where(msk, 1, 0).astype(jnp.int32)
                csum = plsc.cumsum(mi)
                positions = csum - 1 + off
                keep = jnp.logical_and(msk, positions < k)
                jvec = j * 16 + lanes
                plsc.store_scatter(idxrow, [positions], jvec, mask=keep)
                return off + jnp.max(csum)

            off = lax.fori_loop(0, n // 16, chunk, jnp.int32(0))
            offc = jnp.minimum(off, jnp.int32(k))
            plsc.store_scatter(cntbuf, [jnp.full((16,), rl, jnp.int32)],
                               jnp.full((16,), offc, jnp.int32),
                               mask=lanes == 0)
            pltpu.async_copy(table_hbm.at[idxrow], rows_v, sem).wait()
            pltpu.sync_copy(rows_v, g_hbm.at[pl.ds(row * k, k)])
            return carry

        lax.fori_loop(0, rpw, row_body, jnp.int32(0))
        pltpu.sync_copy(cntbuf, cnt_hbm.at[pl.ds(base, rpw)])

    return kfn(d2, table)


def _gmlp_body(g_ref, cnt_ref, pcw_ref, w1_ref, b1_ref, w2_ref, b2_ref,
               w3_ref, b3_ref, o_ref, *, bm, kk):
    wd = g_ref.shape[-1]
    g3 = g_ref[...].reshape(bm, kk, wd)
    pc3 = lax.broadcast_in_dim(pcw_ref[...], (bm, kk, wd), (0, 2))
    g2 = (g3 - pc3).reshape(bm * kk, wd)
    h1 = jnp.maximum(g2 @ w1_ref[...] + b1_ref[...], 0.0)
    h2 = jnp.maximum(h1 @ w2_ref[...] + b2_ref[...], 0.0)
    h3 = h2 @ w3_ref[...] + b3_ref[...]
    h3d = h3.shape[-1]
    h3 = h3.reshape(bm, kk, h3d)
    slot3 = lax.broadcasted_iota(jnp.int32, (bm, kk, h3d), 1)
    cnt3 = lax.broadcast_in_dim(cnt_ref[:, 0:1], (bm, kk, h3d), (0, 1))
    h3 = jnp.where(slot3 < cnt3, h3, -jnp.inf)
    o_ref[...] = jnp.max(h3, axis=1)


def _gmlp_pallas(G, cnt, pcw, w1p, b1, w2, b2, w3, b3, kk=64, bm=128):
    m = cnt.shape[0]
    wd = G.shape[1]
    h1d = w1p.shape[1]
    h2d = w2.shape[1]
    h3d = w3.shape[1]
    body = functools.partial(_gmlp_body, bm=bm, kk=kk)
    out = pl.pallas_call(
        body,
        grid=(m // bm,),
        in_specs=[
            pl.BlockSpec((bm * kk, wd), lambda i: (i, 0)),
            pl.BlockSpec((bm, 128), lambda i: (i, 0)),
            pl.BlockSpec((bm, wd), lambda i: (i, 0)),
            pl.BlockSpec((wd, h1d), lambda i: (0, 0)),
            pl.BlockSpec((h1d,), lambda i: (0,)),
            pl.BlockSpec((h1d, h2d), lambda i: (0, 0)),
            pl.BlockSpec((h2d,), lambda i: (0,)),
            pl.BlockSpec((h2d, h3d), lambda i: (0, 0)),
            pl.BlockSpec((h3d,), lambda i: (0,)),
        ],
        out_specs=pl.BlockSpec((bm, h3d), lambda i: (i, 0)),
        out_shape=jax.ShapeDtypeStruct((m, h3d), jnp.float32),
    )(G, cnt, pcw, w1p, b1, w2, b2, w3, b3)
    return out


def _sa_module(x, posx, posy, posz, m, r, ps, gmlp_bm):
    n = posx.shape[1]
    f = x.shape[1]
    wd = ((f + 3 + 15) // 16) * 16
    rows = 8
    pc128 = _fps_pallas(posx.reshape(rows, n // rows),
                        posy.reshape(rows, n // rows),
                        posz.reshape(rows, n // rows), m)
    d2 = _thresh_pallas(pc128, posx, posy, posz, r * r)
    pos3 = jnp.concatenate([posx, posy, posz], axis=0).T
    table = jnp.zeros((n, wd), jnp.float32)
    table = table.at[:, :f].set(x).at[:, f:f + 3].set(pos3)
    G, cnt = _sc_group(d2, table)
    cnt = jnp.broadcast_to(cnt[:, None], (m, 128))

    (W1, b1), (W2, b2), (W3, b3) = ps
    w1p = jnp.zeros((wd, W1.shape[1]), jnp.float32).at[:f + 3].set(W1)
    pcw = jnp.zeros((m, wd), jnp.float32).at[:, f:f + 3].set(pc128[:, :3])
    out = _gmlp_pallas(G, cnt, pcw, w1p, b1, W2, b2, W3, b3, bm=gmlp_bm)
    return out, pc128


def _mlp(ps, h):
    n = len(ps)
    for i, (W, b) in enumerate(ps):
        h = h @ W + b
        if i < n - 1:
            h = jax.nn.relu(h)
    return h


def _knn_interp(xsrc, pos_src, pos_tgt, k):
    d2 = jnp.sum((pos_tgt[:, None, :] - pos_src[None, :, :]) ** 2, axis=-1)
    _, idx = jax.lax.top_k(-d2, k)
    d2k = jnp.take_along_axis(d2, idx, axis=1)
    w = 1.0 / jnp.maximum(d2k, 1e-16)
    y = jnp.sum(w[:, :, None] * xsrc[idx], axis=1) / jnp.sum(w, axis=1, keepdims=True)
    return y


def _head_body(y_ref, w1_ref, b1_ref, w2_ref, b2_ref, w3_ref, b3_ref, o_ref):
    h = y_ref[...]
    h = jnp.maximum(h @ w1_ref[...] + b1_ref[...], 0.0)
    h = jnp.maximum(h @ w2_ref[...] + b2_ref[...], 0.0)
    o_ref[...] = h @ w3_ref[...] + b3_ref[...]


def _head(y, ps):
    (w1, b1), (w2, b2), (w3, b3) = ps
    R = y.shape[0]
    BR = 1024
    grid = (R // BR,)
    out = pl.pallas_call(
        _head_body,
        grid=grid,
        in_specs=[
            pl.BlockSpec((BR, 128), lambda i: (i, 0)),
            pl.BlockSpec((128, 128), lambda i: (0, 0)),
            pl.BlockSpec((128,), lambda i: (0,)),
            pl.BlockSpec((128, 128), lambda i: (0, 0)),
            pl.BlockSpec((128,), lambda i: (0,)),
            pl.BlockSpec((128, 8), lambda i: (0, 0)),
            pl.BlockSpec((8,), lambda i: (0,)),
        ],
        out_specs=pl.BlockSpec((BR, 8), lambda i: (i, 0)),
        out_shape=jax.ShapeDtypeStruct((R, 8), jnp.float32),
    )(y, w1, b1, w2, b2, w3, b3)
    return out[:, :3]


def kernel(x, pos, batch, params):
    n = pos.shape[0]
    posx = pos[:, 0].reshape(1, n)
    posy = pos[:, 1].reshape(1, n)
    posz = pos[:, 2].reshape(1, n)

    x1, pc1 = _sa_module(x, posx, posy, posz, n // 2, 0.2, params['sa1'],
                         gmlp_bm=128)
    pos1 = pc1[:, :3]
    p1x = pc1[:, 0].reshape(1, n // 2)
    p1y = pc1[:, 1].reshape(1, n // 2)
    p1z = pc1[:, 2].reshape(1, n // 2)
    x2, pc2 = _sa_module(x1, p1x, p1y, p1z, n // 8, 0.4, params['sa2'],
                         gmlp_bm=64)
    pos2 = pc2[:, :3]

    h3 = _mlp(params['sa3'], jnp.concatenate([x2, pos2], axis=-1))
    x3 = jnp.max(h3, axis=0, keepdims=True)
    y = jnp.broadcast_to(x3, (x2.shape[0], x3.shape[1]))
    y = _mlp(params['fp3'], jnp.concatenate([y, x2], axis=-1))
    y = _knn_interp(y, pos2, pos1, 3)
    y = _mlp(params['fp2'], jnp.concatenate([y, x1], axis=-1))
    y = _knn_interp(y, pos1, pos, 3)
    y = _mlp(params['fp1'], jnp.concatenate([y, x], axis=-1))
    (w3, b3) = params['head'][2]
    w3p = jnp.pad(w3, ((0, 0), (0, 5)))
    b3p = jnp.pad(b3, ((0, 5),))
    head_ps = [params['head'][0], params['head'][1], (w3p, b3p)]
    return _head(y, head_ps)


# speedup vs baseline: 11.2901x; 1.5101x over previous
import functools
import jax
import jax.numpy as jnp
import numpy as np
from jax import lax
from jax.experimental import pallas as pl
from jax.experimental.pallas import tpu as pltpu
from jax.experimental.pallas import tpu_sc as plsc


def _fps_body(px_ref, py_ref, pz_ref, pc_ref, *, n, m, cols):
    rows = n // cols
    ri = lax.broadcasted_iota(jnp.int32, (rows, cols), 0)
    ci = lax.broadcasted_iota(jnp.int32, (rows, cols), 1)
    flat = ri * cols + ci
    lane = lax.broadcasted_iota(jnp.int32, (1, 128), 1)

    px = px_ref[...]
    py = py_ref[...]
    pz = pz_ref[...]

    def write_row(i, qx, qy, qz):
        row = (jnp.where(lane == 0, qx, 0.0)
               + jnp.where(lane == 1, qy, 0.0)
               + jnp.where(lane == 2, qz, 0.0))
        pc_ref[pl.ds(i, 1), :] = row

    qx0 = px_ref[0, 0]
    qy0 = py_ref[0, 0]
    qz0 = pz_ref[0, 0]
    write_row(0, qx0, qy0, qz0)

    def body(i, state):
        mind, qx, qy, qz = state
        d = (px - qx) ** 2 + (py - qy) ** 2 + (pz - qz) ** 2
        mind = jnp.minimum(mind, d)
        mval = jnp.max(mind)
        idx = jnp.min(jnp.where(mind == mval, flat, jnp.int32(2 ** 30)))
        sel = flat == idx
        nqx = jnp.sum(jnp.where(sel, px, 0.0))
        nqy = jnp.sum(jnp.where(sel, py, 0.0))
        nqz = jnp.sum(jnp.where(sel, pz, 0.0))
        write_row(i, nqx, nqy, nqz)
        return (mind, nqx, nqy, nqz)

    mind0 = jnp.full((rows, cols), jnp.inf, dtype=jnp.float32)
    lax.fori_loop(1, m, body, (mind0, qx0, qy0, qz0))


def _fps_pallas(posx, posy, posz, m):
    rows, cols = posx.shape
    n = rows * cols
    body = functools.partial(_fps_body, n=n, m=m, cols=cols)
    pc = pl.pallas_call(
        body,
        out_shape=jax.ShapeDtypeStruct((m, 128), jnp.float32),
    )(posx, posy, posz)
    return pc


def _thresh_body(pc_ref, px_ref, py_ref, pz_ref, d2_ref, *, r2, kk, bm):
    cx = pc_ref[:, 0:1]
    cy = pc_ref[:, 1:2]
    cz = pc_ref[:, 2:3]
    d2 = (cx - px_ref[...]) ** 2 + (cy - py_ref[...]) ** 2 + (cz - pz_ref[...]) ** 2

    r2bits = jnp.int32(np.float32(r2).view(np.int32))
    cnt_r = jnp.sum(jnp.where(d2 <= r2, 1.0, 0.0), axis=1, keepdims=True)

    lo0 = jnp.zeros((bm, 1), jnp.int32)
    hi0 = jnp.full((bm, 1), r2bits, jnp.int32)

    def bit_body(_, state):
        lo, hi = state
        mid = lax.div(lo + hi, jnp.int32(2))
        tmid = pltpu.bitcast(mid, jnp.float32)
        cnt = jnp.sum(jnp.where(d2 <= tmid, 1.0, 0.0), axis=1, keepdims=True)
        ge = cnt >= float(kk)
        hi = jnp.where(ge, mid, hi)
        lo = jnp.where(ge, lo, mid + 1)
        return lo, hi

    lo, hi = lax.fori_loop(0, 31, bit_body, (lo0, hi0))
    t = pltpu.bitcast(hi, jnp.float32)
    t = jnp.where(cnt_r >= float(kk), t, jnp.float32(r2))
    d2_ref[...] = d2 - t


def _thresh_pallas(pc128, posx1, posy1, posz1, r2, kk=64, bm=256):
    m = pc128.shape[0]
    n = posx1.shape[1]
    body = functools.partial(_thresh_body, r2=r2, kk=kk, bm=bm)
    d2 = pl.pallas_call(
        body,
        grid=(m // bm,),
        in_specs=[
            pl.BlockSpec((bm, 128), lambda i: (i, 0)),
            pl.BlockSpec((1, n), lambda i: (0, 0)),
            pl.BlockSpec((1, n), lambda i: (0, 0)),
            pl.BlockSpec((1, n), lambda i: (0, 0)),
        ],
        out_specs=pl.BlockSpec((bm, n), lambda i: (i, 0)),
        out_shape=jax.ShapeDtypeStruct((m, n), jnp.float32),
    )(pc128, posx1, posy1, posz1)
    return d2


def _sc_group(d2, table, k=64):
    m, n = d2.shape
    wd = table.shape[1]
    nw = 32
    rpw = m // nw
    mesh = plsc.VectorSubcoreMesh(core_axis_name="c", subcore_axis_name="s")

    @functools.partial(
        pl.kernel,
        mesh=mesh,
        compiler_params=pltpu.CompilerParams(needs_layout_passes=False,
                                             use_tc_tiling_on_sc=False),
        out_type=[
            jax.ShapeDtypeStruct((m * k, wd), jnp.float32),
            jax.ShapeDtypeStruct((m,), jnp.int32),
        ],
        scratch_types=[
            pltpu.VMEM((n,), jnp.float32),
            pltpu.VMEM((k,), jnp.int32),
            pltpu.VMEM((k, wd), jnp.float32),
            pltpu.VMEM((rpw,), jnp.int32),
            pltpu.SemaphoreType.DMA,
        ],
    )
    def kfn(d2_hbm, table_hbm, g_hbm, cnt_hbm,
            d2row, idxrow, rows_v, cntbuf, sem):
        wid = lax.axis_index("s") * 2 + lax.axis_index("c")
        base = wid * rpw
        lanes = lax.broadcasted_iota(jnp.int32, (16,), 0)
        z16 = jnp.zeros((16,), jnp.int32)

        def row_body(rl, carry):
            row = base + rl
            pltpu.sync_copy(d2_hbm.at[row], d2row)
            for q in range(k // 16):
                idxrow[pl.ds(q * 16, 16)] = z16

            def chunk(j, off):
                v = d2row[pl.ds(j * 16, 16)]
                msk = v <= 0.0
                mi = jnp.where(msk, 1, 0).astype(jnp.int32)
                csum = plsc.cumsum(mi)
                positions = csum - 1 + off
                keep = jnp.logical_and(msk, positions < k)
                jvec = j * 16 + lanes
                plsc.store_scatter(idxrow, [positions], jvec, mask=keep)
                return off + jnp.max(csum)

            off = lax.fori_loop(0, n // 16, chunk, jnp.int32(0))
            offc = jnp.minimum(off, jnp.int32(k))
            plsc.store_scatter(cntbuf, [jnp.full((16,), rl, jnp.int32)],
                               jnp.full((16,), offc, jnp.int32),
                               mask=lanes == 0)
            pltpu.async_copy(table_hbm.at[idxrow], rows_v, sem).wait()
            pltpu.sync_copy(rows_v, g_hbm.at[pl.ds(row * k, k)])
            return carry

        lax.fori_loop(0, rpw, row_body, jnp.int32(0))
        pltpu.sync_copy(cntbuf, cnt_hbm.at[pl.ds(base, rpw)])

    return kfn(d2, table)


def _gmlp_body(g_ref, cnt_ref, pcw_ref, w1_ref, b1_ref, w2_ref, b2_ref,
               w3_ref, b3_ref, o_ref, *, bm, kk):
    wd = g_ref.shape[-1]
    g3 = g_ref[...].reshape(bm, kk, wd)
    pc3 = lax.broadcast_in_dim(pcw_ref[...], (bm, kk, wd), (0, 2))
    g2 = (g3 - pc3).reshape(bm * kk, wd)
    h1 = jnp.maximum(g2 @ w1_ref[...] + b1_ref[...], 0.0)
    h2 = jnp.maximum(h1 @ w2_ref[...] + b2_ref[...], 0.0)
    h3 = h2 @ w3_ref[...] + b3_ref[...]
    h3d = h3.shape[-1]
    h3 = h3.reshape(bm, kk, h3d)
    slot3 = lax.broadcasted_iota(jnp.int32, (bm, kk, h3d), 1)
    cnt3 = lax.broadcast_in_dim(cnt_ref[:, 0:1], (bm, kk, h3d), (0, 1))
    h3 = jnp.where(slot3 < cnt3, h3, -jnp.inf)
    o_ref[...] = jnp.max(h3, axis=1)


def _gmlp_pallas(G, cnt, pcw, w1p, b1, w2, b2, w3, b3, kk=64, bm=128):
    m = cnt.shape[0]
    wd = G.shape[1]
    h1d = w1p.shape[1]
    h2d = w2.shape[1]
    h3d = w3.shape[1]
    body = functools.partial(_gmlp_body, bm=bm, kk=kk)
    out = pl.pallas_call(
        body,
        grid=(m // bm,),
        in_specs=[
            pl.BlockSpec((bm * kk, wd), lambda i: (i, 0)),
            pl.BlockSpec((bm, 128), lambda i: (i, 0)),
            pl.BlockSpec((bm, wd), lambda i: (i, 0)),
            pl.BlockSpec((wd, h1d), lambda i: (0, 0)),
            pl.BlockSpec((h1d,), lambda i: (0,)),
            pl.BlockSpec((h1d, h2d), lambda i: (0, 0)),
            pl.BlockSpec((h2d,), lambda i: (0,)),
            pl.BlockSpec((h2d, h3d), lambda i: (0, 0)),
            pl.BlockSpec((h3d,), lambda i: (0,)),
        ],
        out_specs=pl.BlockSpec((bm, h3d), lambda i: (i, 0)),
        out_shape=jax.ShapeDtypeStruct((m, h3d), jnp.float32),
    )(G, cnt, pcw, w1p, b1, w2, b2, w3, b3)
    return out


def _sa_module(x, posx, posy, posz, m, r, ps, gmlp_bm):
    n = posx.shape[1]
    f = x.shape[1]
    wd = ((f + 3 + 15) // 16) * 16
    rows = 8
    pc128 = _fps_pallas(posx.reshape(rows, n // rows),
                        posy.reshape(rows, n // rows),
                        posz.reshape(rows, n // rows), m)
    d2 = _thresh_pallas(pc128, posx, posy, posz, r * r)
    pos3 = jnp.concatenate([posx, posy, posz], axis=0).T
    table = jnp.zeros((n, wd), jnp.float32)
    table = table.at[:, :f].set(x).at[:, f:f + 3].set(pos3)
    G, cnt = _sc_group(d2, table)
    cnt = jnp.broadcast_to(cnt[:, None], (m, 128))

    (W1, b1), (W2, b2), (W3, b3) = ps
    w1p = jnp.zeros((wd, W1.shape[1]), jnp.float32).at[:f + 3].set(W1)
    pcw = jnp.zeros((m, wd), jnp.float32).at[:, f:f + 3].set(pc128[:, :3])
    out = _gmlp_pallas(G, cnt, pcw, w1p, b1, W2, b2, W3, b3, bm=gmlp_bm)
    return out, pc128


def _sa3fp3_body(cat_ref, x2_ref, w1_ref, b1_ref, w2_ref, b2_ref, w3_ref,
                 b3_ref, v1a_ref, v1b_ref, c1_ref, v2_ref, c2_ref, o_ref):
    h = jnp.maximum(cat_ref[...] @ w1_ref[...] + b1_ref[...], 0.0)
    h = jnp.maximum(h @ w2_ref[...] + b2_ref[...], 0.0)
    h = h @ w3_ref[...] + b3_ref[...]
    x3 = jnp.max(h, axis=0, keepdims=True)
    za = x3 @ v1a_ref[...]
    z = jnp.maximum(x2_ref[...] @ v1b_ref[...] + za + c1_ref[...], 0.0)
    o_ref[...] = z @ v2_ref[...] + c2_ref[...]


def _sa3fp3_pallas(cat, x2, sa3_ps, fp3_ps):
    m = cat.shape[0]
    kd = cat.shape[1]
    (w1, b1), (w2, b2), (w3, b3) = sa3_ps
    (v1, c1), (v2, c2) = fp3_ps
    v1a, v1b = v1[:1024], v1[1024:]
    specs = [
        pl.BlockSpec(a.shape, lambda i, r=len(a.shape): (0,) * r)
        for a in (cat, x2, w1, b1, w2, b2, w3, b3, v1a, v1b, c1, v2, c2)
    ]
    out = pl.pallas_call(
        _sa3fp3_body,
        grid=(1,),
        in_specs=specs,
        out_specs=pl.BlockSpec((m, 256), lambda i: (0, 0)),
        out_shape=jax.ShapeDtypeStruct((m, 256), jnp.float32),
    )(cat, x2, w1, b1, w2, b2, w3, b3, v1a, v1b, c1, v2, c2)
    return out


def _interp_sel(pt_ref, sx_ref, sy_ref, sz_ref, bt, ns):
    tx = pt_ref[:, 0:1]
    ty = pt_ref[:, 1:2]
    tz = pt_ref[:, 2:3]
    d2 = (tx - sx_ref[...]) ** 2 + (ty - sy_ref[...]) ** 2 + (tz - sz_ref[...]) ** 2
    ii = lax.broadcasted_iota(jnp.int32, (bt, ns), 1)
    wacc = jnp.zeros((bt, ns), jnp.float32)
    norm = jnp.zeros((bt, 1), jnp.float32)
    dd = d2
    for _ in range(3):
        mv = jnp.min(dd, axis=1, keepdims=True)
        midx = jnp.min(jnp.where(dd == mv, ii, jnp.int32(2 ** 30)), axis=1,
                       keepdims=True)
        hit = ii == midx
        wk = 1.0 / jnp.maximum(mv, 1e-16)
        wacc = wacc + jnp.where(hit, wk, 0.0)
        norm = norm + wk
        dd = jnp.where(hit, jnp.inf, dd)
    return wacc, norm


def _fp2_body(pt_ref, sx_ref, sy_ref, sz_ref, ysrc_ref, x1_ref,
              w1a_ref, w1b_ref, b1_ref, w2_ref, b2_ref, o_ref, *, bt, ns):
    wacc, norm = _interp_sel(pt_ref, sx_ref, sy_ref, sz_ref, bt, ns)
    y = jnp.dot(wacc, ysrc_ref[...], precision=jax.lax.Precision.HIGHEST)
    y = y / norm
    h = jnp.maximum(y @ w1a_ref[...] + x1_ref[...] @ w1b_ref[...] + b1_ref[...], 0.0)
    o_ref[...] = h @ w2_ref[...] + b2_ref[...]


def _fp2_pallas(pc_tgt, sx, sy, sz, ysrc, x1, ps, bt=1024):
    nt = pc_tgt.shape[0]
    ns = sx.shape[1]
    fs = ysrc.shape[1]
    (w1, b1), (w2, b2) = ps
    w1a, w1b = w1[:fs], w1[fs:]
    fskip = w1b.shape[0]
    h2 = w2.shape[1]
    body = functools.partial(_fp2_body, bt=bt, ns=ns)
    out = pl.pallas_call(
        body,
        grid=(nt // bt,),
        in_specs=[
            pl.BlockSpec((bt, 128), lambda i: (i, 0)),
            pl.BlockSpec((1, ns), lambda i: (0, 0)),
            pl.BlockSpec((1, ns), lambda i: (0, 0)),
            pl.BlockSpec((1, ns), lambda i: (0, 0)),
            pl.BlockSpec((ns, fs), lambda i: (0, 0)),
            pl.BlockSpec((bt, fskip), lambda i: (i, 0)),
            pl.BlockSpec(w1a.shape, lambda i: (0, 0)),
            pl.BlockSpec(w1b.shape, lambda i: (0, 0)),
            pl.BlockSpec(b1.shape, lambda i: (0,)),
            pl.BlockSpec(w2.shape, lambda i: (0, 0)),
            pl.BlockSpec(b2.shape, lambda i: (0,)),
        ],
        out_specs=pl.BlockSpec((bt, h2), lambda i: (i, 0)),
        out_shape=jax.ShapeDtypeStruct((nt, h2), jnp.float32),
    )(pc_tgt, sx, sy, sz, ysrc, x1, w1a, w1b, b1, w2, b2)
    return out


def _fp1h_body(pt_ref, sx_ref, sy_ref, sz_ref, ysrc_ref, xs_ref,
               w1a_ref, w1b_ref, b1_ref, w2_ref, b2_ref, w3_ref, b3_ref,
               u1_ref, d1_ref, u2_ref, d2r_ref, u3_ref, d3_ref, o_ref,
               *, bt, ns):
    wacc, norm = _interp_sel(pt_ref, sx_ref, sy_ref, sz_ref, bt, ns)
    y = jnp.dot(wacc, ysrc_ref[...], precision=jax.lax.Precision.HIGHEST)
    y = y / norm
    h = jnp.maximum(y @ w1a_ref[...] + xs_ref[...] @ w1b_ref[...] + b1_ref[...], 0.0)
    h = jnp.maximum(h @ w2_ref[...] + b2_ref[...], 0.0)
    h = h @ w3_ref[...] + b3_ref[...]
    h = jnp.maximum(h @ u1_ref[...] + d1_ref[...], 0.0)
    h = jnp.maximum(h @ u2_ref[...] + d2r_ref[...], 0.0)
    o_ref[...] = h @ u3_ref[...] + d3_ref[...]


def _fp1_head_pallas(pos_tgt, sx, sy, sz, ysrc, xskip, fp1_ps, head_ps,
                     bt=1024):
    nt = pos_tgt.shape[0]
    ptw = pos_tgt.shape[1]
    ns = sx.shape[1]
    fs = ysrc.shape[1]
    (w1, b1), (w2, b2), (w3, b3) = fp1_ps
    (u1, d1), (u2, d2), (u3, d3) = head_ps
    w1a, w1b = w1[:fs], w1[fs:]
    u3p = jnp.pad(u3, ((0, 0), (0, 8 - u3.shape[1])))
    d3p = jnp.pad(d3, ((0, 8 - d3.shape[0]),))
    fskip = w1b.shape[0]
    body = functools.partial(_fp1h_body, bt=bt, ns=ns)
    args = (pos_tgt, sx, sy, sz, ysrc, xskip, w1a, w1b, b1, w2, b2, w3, b3,
            u1, d1, u2, d2, u3p, d3p)
    in_specs = [
        pl.BlockSpec((bt, ptw), lambda i: (i, 0)),
        pl.BlockSpec((1, ns), lambda i: (0, 0)),
        pl.BlockSpec((1, ns), lambda i: (0, 0)),
        pl.BlockSpec((1, ns), lambda i: (0, 0)),
        pl.BlockSpec((ns, fs), lambda i: (0, 0)),
        pl.BlockSpec((bt, fskip), lambda i: (i, 0)),
    ] + [pl.BlockSpec(a.shape, lambda i, r=len(a.shape): (0,) * r)
         for a in args[6:]]
    out = pl.pallas_call(
        body,
        grid=(nt // bt,),
        in_specs=in_specs,
        out_specs=pl.BlockSpec((bt, 8), lambda i: (i, 0)),
        out_shape=jax.ShapeDtypeStruct((nt, 8), jnp.float32),
    )(*args)
    return out[:, :3]


def kernel(x, pos, batch, params):
    n = pos.shape[0]
    posx = pos[:, 0].reshape(1, n)
    posy = pos[:, 1].reshape(1, n)
    posz = pos[:, 2].reshape(1, n)

    x1, pc1 = _sa_module(x, posx, posy, posz, n // 2, 0.2, params['sa1'],
                         gmlp_bm=128)
    p1x = pc1[:, 0].reshape(1, n // 2)
    p1y = pc1[:, 1].reshape(1, n // 2)
    p1z = pc1[:, 2].reshape(1, n // 2)
    x2, pc2 = _sa_module(x1, p1x, p1y, p1z, n // 8, 0.4, params['sa2'],
                         gmlp_bm=64)
    pos2 = pc2[:, :3]

    cat = jnp.concatenate([x2, pos2], axis=-1)
    y3 = _sa3fp3_pallas(cat, x2, params['sa3'], params['fp3'])

    p2x = pc2[:, 0].reshape(1, n // 8)
    p2y = pc2[:, 1].reshape(1, n // 8)
    p2z = pc2[:, 2].reshape(1, n // 8)
    y2 = _fp2_pallas(pc1, p2x, p2y, p2z, y3, x1, params['fp2'])

    return _fp1_head_pallas(pos, p1x, p1y, p1z, y2, x, params['fp1'],
                            params['head'])
